# baseline (speedup 1.0000x reference)
import functools

import jax
import jax.numpy as jnp
from jax import lax
from jax.experimental import pallas as pl
from jax.experimental.pallas import tpu as pltpu
from jax.experimental.pallas import tpu_sc as plsc

K_STEPS = 10
ALPHA = 0.1
HID = 64
NC = 2
NS = 16
F = HID // NS
CHUNKE = 4096
NBUF = 2


def _make_sc_segsum(n_pad, e_pad):
  n_ch = e_pad // (NC * CHUNKE)
  n_pairs = n_ch // NBUF
  assert n_ch % NBUF == 0 and n_pairs >= 2
  n_groups = CHUNKE // 16
  mesh = plsc.VectorSubcoreMesh(core_axis_name="c", subcore_axis_name="s")

  @functools.partial(
      pl.kernel,
      out_type=jax.ShapeDtypeStruct((NC, HID, n_pad), jnp.float32),
      mesh=mesh,
      compiler_params=pltpu.CompilerParams(use_tc_tiling_on_sc=False,
                                           needs_layout_passes=False),
      scratch_types=[
          pltpu.VMEM((F, n_pad), jnp.float32),
          pltpu.VMEM((F, n_pad), jnp.float32),
          [pltpu.VMEM((CHUNKE,), jnp.int32) for _ in range(NBUF)],
          [pltpu.VMEM((CHUNKE,), jnp.int32) for _ in range(NBUF)],
          [pltpu.SemaphoreType.DMA for _ in range(NBUF)],
          [pltpu.SemaphoreType.DMA for _ in range(NBUF)],
      ],
  )
  def sc_segsum(hsT_hbm, row_hbm, col_hbm, out_hbm,
                table_v, acc_v, rbufs, cbufs, rsems, csems):
    c = lax.axis_index("c")
    s = lax.axis_index("s")

    pltpu.sync_copy(hsT_hbm.at[pl.ds(s * F, F)], table_v)

    zeros16 = jnp.zeros((16,), jnp.float32)
    def zrow(i, carry):
      for f in range(F):
        acc_v[f, pl.ds(i * 16, 16)] = zeros16
      return carry
    lax.fori_loop(0, n_pad // 16, zrow, 0)

    def issue(j, b):
      pltpu.async_copy(row_hbm.at[c, j], rbufs[b], rsems[b])
      pltpu.async_copy(col_hbm.at[c, j], cbufs[b], csems[b])

    def wait(b):
      pltpu.make_async_copy(row_hbm.at[c, 0], rbufs[b], rsems[b]).wait()
      pltpu.make_async_copy(col_hbm.at[c, 0], cbufs[b], csems[b]).wait()

    fidx = [jnp.full((16,), f, jnp.int32) for f in range(F)]

    def process(b):
      def subblock(sb, carry):
        @plsc.parallel_loop(0, 16, unroll=8,
                            carry=jnp.zeros((16,), jnp.int32))
        def dupv(g0, dupv):
          g = sb * 16 + g0
          rowv = rbufs[b][pl.ds(g * 16, 16)]
          colv = cbufs[b][pl.ds(g * 16, 16)]
          cnt, _ = plsc.scan_count(colv)
          vals = [plsc.load_gather(table_v, [fidx[f], rowv])
                  for f in range(F)]
          mk = cnt == 1
          for f in range(F):
            plsc.addupdate_scatter(acc_v, [fidx[f], colv], vals[f],
                                   mask=mk)
          return jnp.maximum(dupv, cnt)
        dmax = lax.reduce_max(dupv, (0,))

        @pl.when(dmax > 1)
        def _slow():
          def group2(g0, carry2):
            g = sb * 16 + g0
            rowv = rbufs[b][pl.ds(g * 16, 16)]
            colv = cbufs[b][pl.ds(g * 16, 16)]
            cnt, _ = plsc.scan_count(colv)
            vals = [plsc.load_gather(table_v, [fidx[f], rowv])
                    for f in range(F)]
            dmx = lax.reduce_max(cnt, (0,))
            def kpass(k, carry3):
              mk = cnt == k
              for f in range(F):
                plsc.addupdate_scatter(acc_v, [fidx[f], colv], vals[f],
                                       mask=mk)
              return carry3
            lax.fori_loop(2, dmx + 1, kpass, 0)
            return carry2
          lax.fori_loop(0, 16, group2, 0)
        return carry
      lax.fori_loop(0, n_groups // 16, subblock, 0)

    for b in range(NBUF):
      issue(b, b)

    def pair(j2, carry):
      for b in range(NBUF):
        wait(b)
        process(b)
        issue(j2 * NBUF + b + NBUF, b)
      return carry
    lax.fori_loop(0, n_pairs - 1, pair, 0)
    for b in range(NBUF):
      wait(b)
      process(b)

    pltpu.sync_copy(acc_v, out_hbm.at[c, pl.ds(s * F, F)])

  return sc_segsum


def _tc_prep(x_pad, w1, b1, deg2):
  n_pad = x_pad.shape[0]

  def body(x_ref, w1_ref, b1_ref, d2_ref, hs0_ref, a_ref, rec_ref):
    hpT = lax.dot_general(w1_ref[...], x_ref[...],
                          (((1,), (1,)), ((), ())),
                          preferred_element_type=jnp.float32) + b1_ref[...]
    degT = d2_ref[0] + d2_ref[1] + 1.0
    dinvT = lax.rsqrt(degT)
    hs0_ref[...] = dinvT * hpT
    a_ref[...] = (1.0 - ALPHA) / degT
    rec_ref[...] = jnp.sqrt(degT)

  return pl.pallas_call(
      body,
      out_shape=[jax.ShapeDtypeStruct((HID, n_pad), jnp.float32)] * 3,
  )(x_pad, w1, b1, deg2)


def _tc_update(s2, hs, a, hs0):
  def body(s2_ref, hs_ref, a_ref, hs0_ref, out_ref):
    out_ref[...] = (a_ref[...] * (s2_ref[0] + s2_ref[1] + hs_ref[...])
                    + ALPHA * hs0_ref[...])

  return pl.pallas_call(
      body,
      out_shape=jax.ShapeDtypeStruct(hs.shape, jnp.float32),
  )(s2, hs, a, hs0)


def _tc_out(hs, rec, w2, b2):
  def body(hs_ref, rec_ref, w2_ref, b2_ref, out_ref):
    h = rec_ref[...] * hs_ref[...]
    out_ref[...] = lax.dot_general(h, w2_ref[...], (((0,), (1,)), ((), ())),
                                   preferred_element_type=jnp.float32) + b2_ref[...]

  return pl.pallas_call(
      body,
      out_shape=jax.ShapeDtypeStruct((hs.shape[1], w2.shape[0]), jnp.float32),
  )(hs, rec, w2, b2)


def kernel(x, edge_index, W1, b1, W2, b2):
  n = x.shape[0]
  e = edge_index.shape[1]
  n_pad = ((n + 16 + 127) // 128) * 128 + 128
  egrain = NC * CHUNKE * NBUF
  e_pad = ((e + egrain - 1) // egrain) * egrain

  row = edge_index[0].astype(jnp.int32)
  col = edge_index[1].astype(jnp.int32)
  pad = e_pad - e
  sink = n_pad - 16 + (jnp.arange(pad, dtype=jnp.int32) % 16)
  rowp = jnp.concatenate([row, jnp.zeros((pad,), jnp.int32)])
  colp = jnp.concatenate([col, sink])
  rowp = rowp.reshape(NC, -1, CHUNKE)
  colp = colp.reshape(NC, -1, CHUNKE)

  sc_segsum = _make_sc_segsum(n_pad, e_pad)

  onesT = jnp.ones((HID, n_pad), jnp.float32)
  deg2 = sc_segsum(onesT, rowp, colp)

  x_pad = jnp.pad(x, ((0, n_pad - n), (0, 0)))
  hs0, a, rec = _tc_prep(x_pad, W1, b1.reshape(HID, 1), deg2)

  hs = hs0
  for _ in range(K_STEPS):
    s2 = sc_segsum(hs, rowp, colp)
    hs = _tc_update(s2, hs, a, hs0)

  logits = _tc_out(hs, rec, W2, b2.reshape(1, -1))
  return logits[:n]


# speedup vs baseline: 1.1786x; 1.1786x over previous
import functools

import jax
import jax.numpy as jnp
from jax import lax
from jax.experimental import pallas as pl
from jax.experimental.pallas import tpu as pltpu
from jax.experimental.pallas import tpu_sc as plsc

K_STEPS = 10
ALPHA = 0.1
HID = 64
NC = 2
NS = 16
F = HID // NS
CHUNKE = 4096
NBUF = 2


def _make_sc_segsum(n_pad, e_pad):
  n_ch = e_pad // (NC * CHUNKE)
  n_pairs = n_ch // NBUF
  assert n_ch % NBUF == 0 and n_pairs >= 2
  n_groups = CHUNKE // 16
  mesh = plsc.VectorSubcoreMesh(core_axis_name="c", subcore_axis_name="s")

  @functools.partial(
      pl.kernel,
      out_type=jax.ShapeDtypeStruct((NC, HID, n_pad), jnp.float32),
      mesh=mesh,
      compiler_params=pltpu.CompilerParams(use_tc_tiling_on_sc=False,
                                           needs_layout_passes=False),
      scratch_types=[
          pltpu.VMEM((F, n_pad), jnp.float32),
          pltpu.VMEM((F, n_pad), jnp.float32),
          [pltpu.VMEM((CHUNKE,), jnp.int32) for _ in range(NBUF)],
          [pltpu.VMEM((CHUNKE,), jnp.int32) for _ in range(NBUF)],
          [pltpu.SemaphoreType.DMA for _ in range(NBUF)],
          [pltpu.SemaphoreType.DMA for _ in range(NBUF)],
      ],
  )
  def sc_segsum(hsT_hbm, row_hbm, col_hbm, out_hbm,
                table_v, acc_v, rbufs, cbufs, rsems, csems):
    c = lax.axis_index("c")
    s = lax.axis_index("s")

    pltpu.sync_copy(hsT_hbm.at[pl.ds(s * F, F)], table_v)

    zeros16 = jnp.zeros((16,), jnp.float32)
    def zrow(i, carry):
      for f in range(F):
        acc_v[f, pl.ds(i * 16, 16)] = zeros16
      return carry
    lax.fori_loop(0, n_pad // 16, zrow, 0)

    def issue(j, b):
      pltpu.async_copy(row_hbm.at[c, j], rbufs[b], rsems[b])
      pltpu.async_copy(col_hbm.at[c, j], cbufs[b], csems[b])

    def wait(b):
      pltpu.make_async_copy(row_hbm.at[c, 0], rbufs[b], rsems[b]).wait()
      pltpu.make_async_copy(col_hbm.at[c, 0], cbufs[b], csems[b]).wait()

    fidx = [jnp.full((16,), f, jnp.int32) for f in range(F)]

    def process(b):
      @plsc.parallel_loop(0, n_groups, unroll=16,
                          carry=jnp.zeros((16,), jnp.int32))
      def dupv(g, dupv):
        rowv = rbufs[b][pl.ds(g * 16, 16)]
        colv = cbufs[b][pl.ds(g * 16, 16)]
        cnt, _ = plsc.scan_count(colv)
        vals = [plsc.load_gather(table_v, [fidx[f], rowv])
                for f in range(F)]
        for k in (1, 2):
          mk = cnt == k
          for f in range(F):
            plsc.addupdate_scatter(acc_v, [fidx[f], colv], vals[f],
                                   mask=mk)
        return jnp.maximum(dupv, cnt)
      dmax = lax.reduce_max(dupv, (0,))

      @pl.when(dmax > 2)
      def _slow():
        def group2(g, carry):
          rowv = rbufs[b][pl.ds(g * 16, 16)]
          colv = cbufs[b][pl.ds(g * 16, 16)]
          cnt, _ = plsc.scan_count(colv)
          vals = [plsc.load_gather(table_v, [fidx[f], rowv])
                  for f in range(F)]
          for k in range(3, 17):
            mk = cnt == k
            for f in range(F):
              plsc.addupdate_scatter(acc_v, [fidx[f], colv], vals[f],
                                     mask=mk)
          return carry
        lax.fori_loop(0, n_groups, group2, 0)

    for b in range(NBUF):
      issue(b, b)

    def pair(j2, carry):
      for b in range(NBUF):
        wait(b)
        process(b)
        issue(j2 * NBUF + b + NBUF, b)
      return carry
    lax.fori_loop(0, n_pairs - 1, pair, 0)
    for b in range(NBUF):
      wait(b)
      process(b)

    pltpu.sync_copy(acc_v, out_hbm.at[c, pl.ds(s * F, F)])

  return sc_segsum


def _tc_prep(x_pad, w1, b1, deg2):
  n_pad = x_pad.shape[0]

  def body(x_ref, w1_ref, b1_ref, d2_ref, hs0_ref, a_ref, rec_ref):
    hpT = lax.dot_general(w1_ref[...], x_ref[...],
                          (((1,), (1,)), ((), ())),
                          preferred_element_type=jnp.float32) + b1_ref[...]
    degT = d2_ref[0] + d2_ref[1] + 1.0
    dinvT = lax.rsqrt(degT)
    hs0_ref[...] = dinvT * hpT
    a_ref[...] = (1.0 - ALPHA) / degT
    rec_ref[...] = jnp.sqrt(degT)

  return pl.pallas_call(
      body,
      out_shape=[jax.ShapeDtypeStruct((HID, n_pad), jnp.float32)] * 3,
  )(x_pad, w1, b1, deg2)


def _tc_update(s2, hs, a, hs0):
  def body(s2_ref, hs_ref, a_ref, hs0_ref, out_ref):
    out_ref[...] = (a_ref[...] * (s2_ref[0] + s2_ref[1] + hs_ref[...])
                    + ALPHA * hs0_ref[...])

  return pl.pallas_call(
      body,
      out_shape=jax.ShapeDtypeStruct(hs.shape, jnp.float32),
  )(s2, hs, a, hs0)


def _tc_out(hs, rec, w2, b2):
  def body(hs_ref, rec_ref, w2_ref, b2_ref, out_ref):
    h = rec_ref[...] * hs_ref[...]
    out_ref[...] = lax.dot_general(h, w2_ref[...], (((0,), (1,)), ((), ())),
                                   preferred_element_type=jnp.float32) + b2_ref[...]

  return pl.pallas_call(
      body,
      out_shape=jax.ShapeDtypeStruct((hs.shape[1], w2.shape[0]), jnp.float32),
  )(hs, rec, w2, b2)


def kernel(x, edge_index, W1, b1, W2, b2):
  n = x.shape[0]
  e = edge_index.shape[1]
  n_pad = ((n + 16 + 127) // 128) * 128 + 128
  egrain = NC * CHUNKE * NBUF
  e_pad = ((e + egrain - 1) // egrain) * egrain

  row = edge_index[0].astype(jnp.int32)
  col = edge_index[1].astype(jnp.int32)
  pad = e_pad - e
  sink = n_pad - 16 + (jnp.arange(pad, dtype=jnp.int32) % 16)
  rowp = jnp.concatenate([row, jnp.zeros((pad,), jnp.int32)])
  colp = jnp.concatenate([col, sink])
  rowp = rowp.reshape(NC, -1, CHUNKE)
  colp = colp.reshape(NC, -1, CHUNKE)

  sc_segsum = _make_sc_segsum(n_pad, e_pad)

  onesT = jnp.ones((HID, n_pad), jnp.float32)
  deg2 = sc_segsum(onesT, rowp, colp)

  x_pad = jnp.pad(x, ((0, n_pad - n), (0, 0)))
  hs0, a, rec = _tc_prep(x_pad, W1, b1.reshape(HID, 1), deg2)

  hs = hs0
  for _ in range(K_STEPS):
    s2 = sc_segsum(hs, rowp, colp)
    hs = _tc_update(s2, hs, a, hs0)

  logits = _tc_out(hs, rec, W2, b2.reshape(1, -1))
  return logits[:n]


# speedup vs baseline: 1.4449x; 1.2259x over previous
import functools

import jax
import jax.numpy as jnp
from jax import lax
from jax.experimental import pallas as pl
from jax.experimental.pallas import tpu as pltpu
from jax.experimental.pallas import tpu_sc as plsc

K_STEPS = 10
ALPHA = 0.1
HID = 64
NC = 2
NS = 16
F = HID // NS
CHUNKE = 4096
NBUF = 2


def _make_sc_segsum(n_pad, e_pad):
  n_ch = e_pad // (NC * CHUNKE)
  n_pairs = n_ch // NBUF
  assert n_ch % NBUF == 0 and n_pairs >= 2
  n_groups = CHUNKE // 16
  mesh = plsc.VectorSubcoreMesh(core_axis_name="c", subcore_axis_name="s")

  @functools.partial(
      pl.kernel,
      out_type=jax.ShapeDtypeStruct((NC, HID, n_pad), jnp.float32),
      mesh=mesh,
      compiler_params=pltpu.CompilerParams(use_tc_tiling_on_sc=False,
                                           needs_layout_passes=False),
      scratch_types=[
          pltpu.VMEM((F, n_pad), jnp.float32),
          pltpu.VMEM((F, n_pad), jnp.float32),
          [pltpu.VMEM((CHUNKE,), jnp.int32) for _ in range(NBUF)],
          [pltpu.VMEM((CHUNKE,), jnp.int32) for _ in range(NBUF)],
          [pltpu.SemaphoreType.DMA for _ in range(NBUF)],
          [pltpu.SemaphoreType.DMA for _ in range(NBUF)],
      ],
  )
  def sc_segsum(hsT_hbm, row_hbm, col_hbm, out_hbm,
                table_v, acc_v, rbufs, cbufs, rsems, csems):
    c = lax.axis_index("c")
    s = lax.axis_index("s")

    pltpu.sync_copy(hsT_hbm.at[pl.ds(s * F, F)], table_v)

    zeros16 = jnp.zeros((16,), jnp.float32)
    def zrow(i, carry):
      for f in range(F):
        acc_v[f, pl.ds(i * 16, 16)] = zeros16
      return carry
    lax.fori_loop(0, n_pad // 16, zrow, 0)

    def issue(j, b):
      pltpu.async_copy(row_hbm.at[c, j], rbufs[b], rsems[b])
      pltpu.async_copy(col_hbm.at[c, j], cbufs[b], csems[b])

    def wait(b):
      pltpu.make_async_copy(row_hbm.at[c, 0], rbufs[b], rsems[b]).wait()
      pltpu.make_async_copy(col_hbm.at[c, 0], cbufs[b], csems[b]).wait()

    fidx = [jnp.full((16,), f, jnp.int32) for f in range(F)]

    def process(b):
      @plsc.parallel_loop(0, n_groups, unroll=8,
                          carry=jnp.zeros((16,), jnp.int32))
      def dupv(g, dupv):
        rowv = rbufs[b][pl.ds(g * 16, 16)]
        colv = cbufs[b][pl.ds(g * 16, 16)]
        cnt, _ = plsc.scan_count(colv)
        vals = [plsc.load_gather(table_v.at[f], [rowv])
                for f in range(F)]
        for k in (1, 2):
          mk = cnt == k
          for f in range(F):
            plsc.addupdate_scatter(acc_v.at[f], [colv], vals[f],
                                   mask=mk)
        return jnp.maximum(dupv, cnt)
      dmax = lax.reduce_max(dupv, (0,))

      @pl.when(dmax > 2)
      def _slow():
        def group2(g, carry):
          rowv = rbufs[b][pl.ds(g * 16, 16)]
          colv = cbufs[b][pl.ds(g * 16, 16)]
          cnt, _ = plsc.scan_count(colv)
          vals = [plsc.load_gather(table_v.at[f], [rowv])
                  for f in range(F)]
          for k in range(3, 17):
            mk = cnt == k
            for f in range(F):
              plsc.addupdate_scatter(acc_v.at[f], [colv], vals[f],
                                     mask=mk)
          return carry
        lax.fori_loop(0, n_groups, group2, 0)

    for b in range(NBUF):
      issue(b, b)

    def pair(j2, carry):
      for b in range(NBUF):
        wait(b)
        process(b)
        issue(j2 * NBUF + b + NBUF, b)
      return carry
    lax.fori_loop(0, n_pairs - 1, pair, 0)
    for b in range(NBUF):
      wait(b)
      process(b)

    pltpu.sync_copy(acc_v, out_hbm.at[c, pl.ds(s * F, F)])

  return sc_segsum


def _tc_prep(x_pad, w1, b1, deg2):
  n_pad = x_pad.shape[0]

  def body(x_ref, w1_ref, b1_ref, d2_ref, hs0_ref, a_ref, rec_ref):
    hpT = lax.dot_general(w1_ref[...], x_ref[...],
                          (((1,), (1,)), ((), ())),
                          preferred_element_type=jnp.float32) + b1_ref[...]
    degT = d2_ref[0] + d2_ref[1] + 1.0
    dinvT = lax.rsqrt(degT)
    hs0_ref[...] = dinvT * hpT
    a_ref[...] = (1.0 - ALPHA) / degT
    rec_ref[...] = jnp.sqrt(degT)

  return pl.pallas_call(
      body,
      out_shape=[jax.ShapeDtypeStruct((HID, n_pad), jnp.float32)] * 3,
  )(x_pad, w1, b1, deg2)


def _tc_update(s2, hs, a, hs0):
  def body(s2_ref, hs_ref, a_ref, hs0_ref, out_ref):
    out_ref[...] = (a_ref[...] * (s2_ref[0] + s2_ref[1] + hs_ref[...])
                    + ALPHA * hs0_ref[...])

  return pl.pallas_call(
      body,
      out_shape=jax.ShapeDtypeStruct(hs.shape, jnp.float32),
  )(s2, hs, a, hs0)


def _tc_out(hs, rec, w2, b2):
  def body(hs_ref, rec_ref, w2_ref, b2_ref, out_ref):
    h = rec_ref[...] * hs_ref[...]
    out_ref[...] = lax.dot_general(h, w2_ref[...], (((0,), (1,)), ((), ())),
                                   preferred_element_type=jnp.float32) + b2_ref[...]

  return pl.pallas_call(
      body,
      out_shape=jax.ShapeDtypeStruct((hs.shape[1], w2.shape[0]), jnp.float32),
  )(hs, rec, w2, b2)


def kernel(x, edge_index, W1, b1, W2, b2):
  n = x.shape[0]
  e = edge_index.shape[1]
  n_pad = ((n + 16 + 127) // 128) * 128 + 128
  egrain = NC * CHUNKE * NBUF
  e_pad = ((e + egrain - 1) // egrain) * egrain

  row = edge_index[0].astype(jnp.int32)
  col = edge_index[1].astype(jnp.int32)
  pad = e_pad - e
  sink = n_pad - 16 + (jnp.arange(pad, dtype=jnp.int32) % 16)
  rowp = jnp.concatenate([row, jnp.zeros((pad,), jnp.int32)])
  colp = jnp.concatenate([col, sink])
  rowp = rowp.reshape(NC, -1, CHUNKE)
  colp = colp.reshape(NC, -1, CHUNKE)

  sc_segsum = _make_sc_segsum(n_pad, e_pad)

  onesT = jnp.ones((HID, n_pad), jnp.float32)
  deg2 = sc_segsum(onesT, rowp, colp)

  x_pad = jnp.pad(x, ((0, n_pad - n), (0, 0)))
  hs0, a, rec = _tc_prep(x_pad, W1, b1.reshape(HID, 1), deg2)

  hs = hs0
  for _ in range(K_STEPS):
    s2 = sc_segsum(hs, rowp, colp)
    hs = _tc_update(s2, hs, a, hs0)

  logits = _tc_out(hs, rec, W2, b2.reshape(1, -1))
  return logits[:n]
